# baseline (speedup 1.0000x reference)
import jax
import jax.numpy as jnp
from jax.experimental import pallas as pl
from jax.experimental.pallas import tpu as pltpu

N = 10000
F_IN = 128
F_HID = 64
NBUF = 3
BMAX = 320

_SCHEDULE = [40, 40, 80, 160] + [320] * 29 + [240, 120, 40]
_STARTS = [sum(_SCHEDULE[:i]) for i in range(len(_SCHEDULE))]
assert sum(_SCHEDULE) == N


def _gcn_kernel(x_ref, w_ref, b_ref, adj_ref, logp_ref, embed_ref,
                buf_ref, support_ref, sem_ref):
    def start_copy(k):
        slot = k % NBUF
        rows = _SCHEDULE[k]
        pltpu.make_async_copy(
            adj_ref.at[pl.ds(_STARTS[k], rows), :],
            buf_ref.at[slot, pl.ds(0, rows), :],
            sem_ref.at[slot],
        ).start()

    def wait_copy(k):
        slot = k % NBUF
        rows = _SCHEDULE[k]
        pltpu.make_async_copy(
            adj_ref.at[pl.ds(_STARTS[k], rows), :],
            buf_ref.at[slot, pl.ds(0, rows), :],
            sem_ref.at[slot],
        ).wait()

    for k in range(NBUF):
        start_copy(k)

    support_ref[:] = jnp.dot(
        x_ref[:], w_ref[:], preferred_element_type=jnp.float32
    )

    for k, rows in enumerate(_SCHEDULE):
        slot = k % NBUF
        wait_copy(k)
        out = jnp.dot(buf_ref[slot, 0:rows, :], support_ref[:],
                      preferred_element_type=jnp.float32)
        out = out + b_ref[:]
        embed_ref[pl.ds(_STARTS[k], rows), :] = out
        m = jnp.max(out, axis=1, keepdims=True)
        lse = jnp.log(jnp.sum(jnp.exp(out - m), axis=1, keepdims=True)) + m
        logp_ref[pl.ds(_STARTS[k], rows), :] = out - lse
        if k + NBUF < len(_SCHEDULE):
            start_copy(k + NBUF)


def kernel(x, adj, W, b):
    b2 = b.reshape(1, F_HID)
    logp, embed = pl.pallas_call(
        _gcn_kernel,
        in_specs=[
            pl.BlockSpec(memory_space=pltpu.VMEM),
            pl.BlockSpec(memory_space=pltpu.VMEM),
            pl.BlockSpec(memory_space=pltpu.VMEM),
            pl.BlockSpec(memory_space=pl.ANY),
        ],
        out_specs=[
            pl.BlockSpec(memory_space=pltpu.VMEM),
            pl.BlockSpec(memory_space=pltpu.VMEM),
        ],
        out_shape=[
            jax.ShapeDtypeStruct((N, F_HID), jnp.float32),
            jax.ShapeDtypeStruct((N, F_HID), jnp.float32),
        ],
        scratch_shapes=[
            pltpu.VMEM((NBUF, BMAX, N), jnp.float32),
            pltpu.VMEM((N, F_HID), jnp.float32),
            pltpu.SemaphoreType.DMA((NBUF,)),
        ],
    )(x, W, b2, adj)
    return (logp, embed)
